# Initial kernel scaffold; baseline (speedup 1.0000x reference)
import jax
import jax.numpy as jnp
from jax.experimental import pallas as pl


def kernel(x, edge_index, edge_attr, batch, fc_in_w, fc_in_b, ln_g0, ln_b0, lin_edge_w0, lin_edge_b0, t0, mlp_w10, mlp_b10, bn_g0, bn_b0, bn_rm0, bn_rv0, mlp_w20, mlp_b20, ln_g1, ln_b1, lin_edge_w1, lin_edge_b1, t1, mlp_w11, mlp_b11, bn_g1, bn_b1, bn_rm1, bn_rv1, mlp_w21, mlp_b21, fc_out_w, fc_out_b):
    raise NotImplementedError("write your pallas kernel here")


# speedup vs baseline: 5.1690x; 5.1690x over previous
import functools

import jax
import jax.numpy as jnp
from jax import lax
from jax.experimental import pallas as pl
from jax.experimental.pallas import tpu as pltpu
from jax.experimental.pallas import tpu_sc as plsc

N = 10000
E = 320000
D = 128
ED = 16
G = 16
GEN_EPS = 1e-7
BN_EPS = 1e-5
LN_EPS = 1e-5

NC = 2
NS = 16
NW = NC * NS
CH = 80
KSUB = 5
MCH = CH * KSUB
KSUB_S = 2
MCH_S = CH * KSUB_S

BN_ROWS = 1000
BE = 2000

_vmesh = plsc.VectorSubcoreMesh(core_axis_name="c", subcore_axis_name="s")


def _absmax_body(a_ref, o_ref):
    i = pl.program_id(0)

    @pl.when(i == 0)
    def _():
        o_ref[0, 0] = 0.0

    o_ref[0, 0] = jnp.maximum(o_ref[0, 0], jnp.max(jnp.abs(a_ref[...])))


def _edge_attr_absmax(edge_attr):
    bea = 4000
    return pl.pallas_call(
        _absmax_body,
        grid=(E // bea,),
        in_specs=[pl.BlockSpec((bea, ED), lambda i: (i, 0))],
        out_specs=pl.BlockSpec(memory_space=pltpu.SMEM),
        out_shape=jax.ShapeDtypeStruct((1, 1), jnp.float32),
    )(edge_attr)


def _fcin_body(x_ref, wt_ref, b_ref, g_ref, bb_ref, x1_ref, h_ref):
    y = jnp.dot(x_ref[...], wt_ref[...], preferred_element_type=jnp.float32)
    y = y + b_ref[...]
    x1_ref[...] = y
    mu = jnp.mean(y, axis=1, keepdims=True)
    var = jnp.mean((y - mu) ** 2, axis=1, keepdims=True)
    hn = (y - mu) * lax.rsqrt(var + LN_EPS) * g_ref[...] + bb_ref[...]
    h_ref[...] = jnp.maximum(hn, 0.0)


def _fcin_ln(x, wt, b, g, bb):
    return pl.pallas_call(
        _fcin_body,
        grid=(N // BN_ROWS,),
        in_specs=[
            pl.BlockSpec((BN_ROWS, D), lambda i: (i, 0)),
            pl.BlockSpec((D, D), lambda i: (0, 0)),
            pl.BlockSpec((1, D), lambda i: (0, 0)),
            pl.BlockSpec((1, D), lambda i: (0, 0)),
            pl.BlockSpec((1, D), lambda i: (0, 0)),
        ],
        out_specs=[
            pl.BlockSpec((BN_ROWS, D), lambda i: (i, 0)),
            pl.BlockSpec((BN_ROWS, D), lambda i: (i, 0)),
        ],
        out_shape=[
            jax.ShapeDtypeStruct((N, D), jnp.float32),
            jax.ShapeDtypeStruct((N, D), jnp.float32),
        ],
    )(x, wt, b.reshape(1, D), g.reshape(1, D), bb.reshape(1, D))


def _sc_gather(h, src):
    epw = E // NW
    nmc = epw // MCH

    @functools.partial(
        pl.kernel,
        out_type=jax.ShapeDtypeStruct((E, D), jnp.float32),
        mesh=_vmesh,
        scratch_types=[
            pltpu.VMEM((MCH,), jnp.int32),
            pltpu.VMEM((MCH, D), jnp.float32),
            pltpu.SemaphoreType.DMA,
        ],
    )
    def k(h_hbm, src_hbm, out_hbm, idx_v, rows_v, sem):
        cid = lax.axis_index("c")
        sid = lax.axis_index("s")
        wid = sid * NC + cid
        base = wid * epw

        @pl.loop(0, nmc)
        def _(mc):
            eb = base + mc * MCH
            pltpu.sync_copy(src_hbm.at[pl.ds(eb, MCH)], idx_v)
            for kk in range(KSUB):
                pltpu.async_copy(
                    h_hbm.at[idx_v.at[pl.ds(kk * CH, CH)]],
                    rows_v.at[pl.ds(kk * CH, CH)],
                    sem,
                ).wait()
            pltpu.sync_copy(rows_v, out_hbm.at[pl.ds(eb, MCH)])

    return k(h, src)


def _mid_body(t_ref, c_ref, g_ref, a_ref, lewt_ref, leb_ref, o_ref):
    e = jnp.dot(a_ref[...], lewt_ref[...], preferred_element_type=jnp.float32)
    e = e + leb_ref[...]
    m = jnp.maximum(g_ref[...] + e, 0.0) + GEN_EPS
    ex = jnp.exp(m * t_ref[0, 0] - c_ref[0, 0])
    o_ref[0] = m * ex
    o_ref[1] = ex


def _mid(t, c, gathered, edge_attr, lewt, leb):
    return pl.pallas_call(
        _mid_body,
        grid=(E // BE,),
        in_specs=[
            pl.BlockSpec(memory_space=pltpu.SMEM),
            pl.BlockSpec(memory_space=pltpu.SMEM),
            pl.BlockSpec((BE, D), lambda i: (i, 0)),
            pl.BlockSpec((BE, ED), lambda i: (i, 0)),
            pl.BlockSpec((ED, D), lambda i: (0, 0)),
            pl.BlockSpec((1, D), lambda i: (0, 0)),
        ],
        out_specs=pl.BlockSpec((2, BE, D), lambda i: (0, i, 0)),
        out_shape=jax.ShapeDtypeStruct((2, E, D), jnp.float32),
    )(t.reshape(1, 1), c.reshape(1, 1), gathered, edge_attr, lewt,
      leb.reshape(1, D))


def _sc_scatter(vals, dst, zeros):
    epc = E // NS
    nmc = epc // MCH_S
    rpt = (N // NS) // 8 * 8
    rem = N - rpt * NS

    @functools.partial(
        pl.kernel,
        out_type=jax.ShapeDtypeStruct((NC, N, D), jnp.float32),
        mesh=_vmesh,
        scratch_types=[
            pltpu.VMEM_SHARED((N, D), jnp.float32),
        ] + [pltpu.VMEM((CH,), jnp.int32) for _ in range(KSUB_S)] + [
            pltpu.VMEM((MCH_S, D), jnp.float32),
            pltpu.SemaphoreType.DMA,
        ],
    )
    def k(vals_hbm, dst_hbm, z_hbm, out_hbm, acc, *rest):
        idx_bufs = rest[:KSUB_S]
        rows_v = rest[KSUB_S]
        cid = lax.axis_index("c")
        sid = lax.axis_index("s")
        pltpu.sync_copy(z_hbm.at[pl.ds(sid * rpt, rpt)],
                        acc.at[pl.ds(sid * rpt, rpt)])

        @pl.when(sid == 0)
        def _():
            pltpu.sync_copy(z_hbm.at[pl.ds(rpt * NS, rem)],
                            acc.at[pl.ds(rpt * NS, rem)])

        plsc.subcore_barrier()
        base = sid * epc

        @pl.loop(0, nmc)
        def _(mc):
            eb = base + mc * MCH_S
            pltpu.sync_copy(vals_hbm.at[cid].at[pl.ds(eb, MCH_S)], rows_v)
            for kk in range(KSUB_S):
                pltpu.sync_copy(dst_hbm.at[pl.ds(eb + kk * CH, CH)],
                                idx_bufs[kk])
            for kk in range(KSUB_S):
                pltpu.sync_copy(
                    rows_v.at[pl.ds(kk * CH, CH)],
                    acc.at[idx_bufs[kk]],
                    add=True,
                )

        plsc.subcore_barrier()
        pltpu.sync_copy(acc.at[pl.ds(sid * rpt, rpt)],
                        out_hbm.at[cid].at[pl.ds(sid * rpt, rpt)])

        @pl.when(sid == 0)
        def _():
            pltpu.sync_copy(acc.at[pl.ds(rpt * NS, rem)],
                            out_hbm.at[cid].at[pl.ds(rpt * NS, rem)])

    return k(vals, dst, zeros)


def _tail(x, h, num, den, w1t, s1, q1, w2t, b2, lng, lnb, emit_h):
    def body(x_ref, h_ref, num_ref, den_ref, w1t_ref, s1_ref, q1_ref,
             w2t_ref, b2_ref, g_ref, bb_ref, xn_ref, *maybe_hn):
        dn = den_ref[...]
        safe = jnp.where(dn > 0.0, dn, 1.0)
        agg = jnp.where(dn > 0.0, num_ref[...] / safe, 0.0)
        out = agg + h_ref[...]
        z = jnp.dot(out, w1t_ref[...], preferred_element_type=jnp.float32)
        z = jnp.maximum(z * s1_ref[...] + q1_ref[...], 0.0)
        y = jnp.dot(z, w2t_ref[...], preferred_element_type=jnp.float32)
        xn = x_ref[...] + y + b2_ref[...]
        xn_ref[...] = xn
        if emit_h:
            mu = jnp.mean(xn, axis=1, keepdims=True)
            var = jnp.mean((xn - mu) ** 2, axis=1, keepdims=True)
            hn = (xn - mu) * lax.rsqrt(var + LN_EPS) * g_ref[...] + bb_ref[...]
            maybe_hn[0][...] = jnp.maximum(hn, 0.0)

    n_out = 2 if emit_h else 1
    row = lambda i: (i, 0)
    full = lambda i: (0, 0)
    res = pl.pallas_call(
        body,
        grid=(N // BN_ROWS,),
        in_specs=[
            pl.BlockSpec((BN_ROWS, D), row),
            pl.BlockSpec((BN_ROWS, D), row),
            pl.BlockSpec((BN_ROWS, D), row),
            pl.BlockSpec((BN_ROWS, D), row),
            pl.BlockSpec((D, 2 * D), full),
            pl.BlockSpec((1, 2 * D), full),
            pl.BlockSpec((1, 2 * D), full),
            pl.BlockSpec((2 * D, D), full),
            pl.BlockSpec((1, D), full),
            pl.BlockSpec((1, D), full),
            pl.BlockSpec((1, D), full),
        ],
        out_specs=[pl.BlockSpec((BN_ROWS, D), row)] * n_out,
        out_shape=[jax.ShapeDtypeStruct((N, D), jnp.float32)] * n_out,
    )(x, h, num, den, w1t, s1.reshape(1, 2 * D), q1.reshape(1, 2 * D),
      w2t, b2.reshape(1, D), lng.reshape(1, D), lnb.reshape(1, D))
    return (res[0], res[1]) if emit_h else (res[0], None)


def _pool_body(bid_ref, x_ref, fot_ref, fob_ref, o_ref, acc, cnt):
    i = pl.program_id(0)

    @pl.when(i == 0)
    def _():
        acc[...] = jnp.zeros_like(acc)
        cnt[...] = jnp.zeros_like(cnt)

    bids = bid_ref[0]
    gid = lax.broadcasted_iota(jnp.int32, (G, BN_ROWS), 0)
    oh = (gid == bids).astype(jnp.float32)
    acc[...] += jnp.dot(oh, x_ref[...], preferred_element_type=jnp.float32)
    cnt[...] = cnt[...] + jnp.sum(oh, axis=1, keepdims=True)

    @pl.when(i == pl.num_programs(0) - 1)
    def _():
        pooled = acc[...] / jnp.maximum(cnt[...], 1.0)
        pooled = jnp.maximum(pooled, 0.0)
        o_ref[...] = (
            jnp.dot(pooled, fot_ref[...], preferred_element_type=jnp.float32)
            + fob_ref[...]
        )


def _pool(batch3, x, fot, fob):
    return pl.pallas_call(
        _pool_body,
        grid=(N // BN_ROWS,),
        in_specs=[
            pl.BlockSpec((1, 1, BN_ROWS), lambda i: (i, 0, 0)),
            pl.BlockSpec((BN_ROWS, D), lambda i: (i, 0)),
            pl.BlockSpec((D, D), lambda i: (0, 0)),
            pl.BlockSpec((1, D), lambda i: (0, 0)),
        ],
        out_specs=pl.BlockSpec((G, D), lambda i: (0, 0)),
        out_shape=jax.ShapeDtypeStruct((G, D), jnp.float32),
        scratch_shapes=[
            pltpu.VMEM((G, D), jnp.float32),
            pltpu.VMEM((G, D), jnp.float32),
        ],
    )(batch3, x, fot, fob.reshape(1, D))


def kernel(x, edge_index, edge_attr, batch, fc_in_w, fc_in_b,
           ln_g0, ln_b0, lin_edge_w0, lin_edge_b0, t0, mlp_w10, mlp_b10,
           bn_g0, bn_b0, bn_rm0, bn_rv0, mlp_w20, mlp_b20,
           ln_g1, ln_b1, lin_edge_w1, lin_edge_b1, t1, mlp_w11, mlp_b11,
           bn_g1, bn_b1, bn_rm1, bn_rv1, mlp_w21, mlp_b21,
           fc_out_w, fc_out_b):
    src = edge_index[0]
    dst = edge_index[1]
    batch3 = batch.reshape(N // BN_ROWS, 1, BN_ROWS)
    zeros = jnp.zeros((N, D), jnp.float32)

    amax = _edge_attr_absmax(edge_attr)[0, 0]

    x1, h = _fcin_ln(x, fc_in_w.T, fc_in_b, ln_g0, ln_b0)

    layers = [
        (ln_g0, ln_b0, lin_edge_w0, lin_edge_b0, t0, mlp_w10, mlp_b10,
         bn_g0, bn_b0, bn_rm0, bn_rv0, mlp_w20, mlp_b20, ln_g1, ln_b1),
        (ln_g1, ln_b1, lin_edge_w1, lin_edge_b1, t1, mlp_w11, mlp_b11,
         bn_g1, bn_b1, bn_rm1, bn_rv1, mlp_w21, mlp_b21, None, None),
    ]
    for l, (lng, lnb, lew, leb, t, w1, b1, bg, bb, brm, brv, w2, b2,
            lng_next, lnb_next) in enumerate(layers):
        srow = jnp.max(jnp.sum(jnp.abs(lew), axis=1))
        bound = (11.32 * jnp.max(jnp.abs(lng)) + jnp.max(jnp.abs(lnb))
                 + amax * srow + jnp.max(jnp.abs(leb)))
        c = jnp.minimum(jnp.abs(t) * bound + 1.0, 80.0)

        gathered = _sc_gather(h, src)
        vals = _mid(t, c, gathered, edge_attr, lew.T, leb)
        nd = _sc_scatter(vals, dst, zeros)

        s1 = bg * lax.rsqrt(brv + BN_EPS)
        q1 = (b1 - brm) * s1 + bb
        if l == 0:
            x1, h = _tail(x1, h, nd[0], nd[1], w1.T, s1, q1, w2.T, b2,
                          lng_next, lnb_next, emit_h=True)
        else:
            x1, _ = _tail(x1, h, nd[0], nd[1], w1.T, s1, q1, w2.T, b2,
                          lng, lnb, emit_h=False)

    return _pool(batch3, x1, fc_out_w.T, fc_out_b)


# speedup vs baseline: 7.2886x; 1.4101x over previous
import functools

import jax
import jax.numpy as jnp
from jax import lax
from jax.experimental import pallas as pl
from jax.experimental.pallas import tpu as pltpu
from jax.experimental.pallas import tpu_sc as plsc

N = 10000
E = 320000
D = 128
ED = 16
G = 16
GEN_EPS = 1e-7
BN_EPS = 1e-5
LN_EPS = 1e-5

NC = 2
NS = 16
NW = NC * NS
CH = 80
KSUB = 5
MCH = CH * KSUB
KSUB_S = 2
MCH_S = CH * KSUB_S

BN_ROWS = 1000
BE = 2000

_vmesh = plsc.VectorSubcoreMesh(core_axis_name="c", subcore_axis_name="s")


def _absmax_body(a_ref, o_ref):
    i = pl.program_id(0)

    @pl.when(i == 0)
    def _():
        o_ref[0, 0] = 0.0

    o_ref[0, 0] = jnp.maximum(o_ref[0, 0], jnp.max(jnp.abs(a_ref[...])))


def _edge_attr_absmax(edge_attr):
    bea = 4000
    return pl.pallas_call(
        _absmax_body,
        grid=(E // bea,),
        in_specs=[pl.BlockSpec((bea, ED), lambda i: (i, 0))],
        out_specs=pl.BlockSpec(memory_space=pltpu.SMEM),
        out_shape=jax.ShapeDtypeStruct((1, 1), jnp.float32),
    )(edge_attr)


def _fcin_body(x_ref, wt_ref, b_ref, g_ref, bb_ref, x1_ref, h_ref):
    y = jnp.dot(x_ref[...], wt_ref[...], preferred_element_type=jnp.float32)
    y = y + b_ref[...]
    x1_ref[...] = y
    mu = jnp.mean(y, axis=1, keepdims=True)
    var = jnp.mean((y - mu) ** 2, axis=1, keepdims=True)
    hn = (y - mu) * lax.rsqrt(var + LN_EPS) * g_ref[...] + bb_ref[...]
    h_ref[...] = jnp.maximum(hn, 0.0)


def _fcin_ln(x, wt, b, g, bb):
    return pl.pallas_call(
        _fcin_body,
        grid=(N // BN_ROWS,),
        in_specs=[
            pl.BlockSpec((BN_ROWS, D), lambda i: (i, 0)),
            pl.BlockSpec((D, D), lambda i: (0, 0)),
            pl.BlockSpec((1, D), lambda i: (0, 0)),
            pl.BlockSpec((1, D), lambda i: (0, 0)),
            pl.BlockSpec((1, D), lambda i: (0, 0)),
        ],
        out_specs=[
            pl.BlockSpec((BN_ROWS, D), lambda i: (i, 0)),
            pl.BlockSpec((BN_ROWS, D), lambda i: (i, 0)),
        ],
        out_shape=[
            jax.ShapeDtypeStruct((N, D), jnp.float32),
            jax.ShapeDtypeStruct((N, D), jnp.float32),
        ],
    )(x, wt, b.reshape(1, D), g.reshape(1, D), bb.reshape(1, D))


def _sc_gather(h, src):
    epw = E // NW
    nmc = epw // MCH

    @functools.partial(
        pl.kernel,
        out_type=jax.ShapeDtypeStruct((E, D), jnp.float32),
        mesh=_vmesh,
        scratch_types=[
            pltpu.VMEM((MCH,), jnp.int32),
            pltpu.VMEM((MCH,), jnp.int32),
            pltpu.VMEM((MCH, D), jnp.float32),
            pltpu.VMEM((MCH, D), jnp.float32),
        ] + [pltpu.SemaphoreType.DMA] * 6,
    )
    def k(h_hbm, src_hbm, out_hbm, idx_a, idx_b, rows_a, rows_b,
          si_a, si_b, sg_a, sg_b, sw_a, sw_b):
        cid = lax.axis_index("c")
        sid = lax.axis_index("s")
        wid = sid * NC + cid
        base = wid * epw
        slots = ((idx_a, rows_a, si_a, sg_a, sw_a),
                 (idx_b, rows_b, si_b, sg_b, sw_b))

        def start_idx(slot, ch):
            idx_v, _, si, _, _ = slot
            pltpu.async_copy(src_hbm.at[pl.ds(base + ch * MCH, MCH)],
                             idx_v, si)

        def process(slot, ch):
            idx_v, rows_v, si, sg, sw = slot
            eb = base + ch * MCH
            pltpu.make_async_copy(src_hbm.at[pl.ds(eb, MCH)], idx_v, si).wait()

            @pl.when(ch >= 2)
            def _():
                pltpu.make_async_copy(rows_v, out_hbm.at[pl.ds(eb, MCH)],
                                      sw).wait()

            for kk in range(KSUB):
                pltpu.async_copy(
                    h_hbm.at[idx_v.at[pl.ds(kk * CH, CH)]],
                    rows_v.at[pl.ds(kk * CH, CH)],
                    sg,
                )
            for kk in range(KSUB):
                pltpu.make_async_copy(
                    h_hbm.at[idx_v.at[pl.ds(kk * CH, CH)]],
                    rows_v.at[pl.ds(kk * CH, CH)],
                    sg,
                ).wait()
            pltpu.async_copy(rows_v, out_hbm.at[pl.ds(eb, MCH)], sw)

        start_idx(slots[0], 0)

        @pl.loop(0, nmc - 1, step=2)
        def _(g):
            start_idx(slots[1], g + 1)
            process(slots[0], g)
            start_idx(slots[0], g + 2)
            process(slots[1], g + 1)

        process(slots[0], nmc - 1)
        pltpu.make_async_copy(rows_a, out_hbm.at[pl.ds(base, MCH)], sw_a).wait()
        pltpu.make_async_copy(rows_b, out_hbm.at[pl.ds(base, MCH)], sw_b).wait()

    return k(h, src)


def _mid_body(t_ref, c_ref, g_ref, a_ref, lewt_ref, leb_ref, o_ref):
    e = jnp.dot(a_ref[...], lewt_ref[...], preferred_element_type=jnp.float32)
    e = e + leb_ref[...]
    m = jnp.maximum(g_ref[...] + e, 0.0) + GEN_EPS
    ex = jnp.exp(m * t_ref[0, 0] - c_ref[0, 0])
    o_ref[0] = m * ex
    o_ref[1] = ex


def _mid(t, c, gathered, edge_attr, lewt, leb):
    return pl.pallas_call(
        _mid_body,
        grid=(E // BE,),
        in_specs=[
            pl.BlockSpec(memory_space=pltpu.SMEM),
            pl.BlockSpec(memory_space=pltpu.SMEM),
            pl.BlockSpec((BE, D), lambda i: (i, 0)),
            pl.BlockSpec((BE, ED), lambda i: (i, 0)),
            pl.BlockSpec((ED, D), lambda i: (0, 0)),
            pl.BlockSpec((1, D), lambda i: (0, 0)),
        ],
        out_specs=pl.BlockSpec((2, BE, D), lambda i: (0, i, 0)),
        out_shape=jax.ShapeDtypeStruct((2, E, D), jnp.float32),
    )(t.reshape(1, 1), c.reshape(1, 1), gathered, edge_attr, lewt,
      leb.reshape(1, D))


def _sc_scatter(vals, dst, zeros):
    epc = E // NS
    nmc = epc // MCH_S
    rpt = (N // NS) // 8 * 8
    rem = N - rpt * NS

    @functools.partial(
        pl.kernel,
        out_type=jax.ShapeDtypeStruct((NC, N, D), jnp.float32),
        mesh=_vmesh,
        scratch_types=[
            pltpu.VMEM_SHARED((N, D), jnp.float32),
        ] + [pltpu.VMEM((CH,), jnp.int32) for _ in range(2 * KSUB_S)] + [
            pltpu.VMEM((MCH_S, D), jnp.float32),
            pltpu.VMEM((MCH_S, D), jnp.float32),
            pltpu.SemaphoreType.DMA,
            pltpu.SemaphoreType.DMA,
        ],
    )
    def k(vals_hbm, dst_hbm, z_hbm, out_hbm, acc, *rest):
        idx_a = rest[:KSUB_S]
        idx_b = rest[KSUB_S:2 * KSUB_S]
        rows_a, rows_b, sem_a, sem_b = rest[2 * KSUB_S:]
        cid = lax.axis_index("c")
        sid = lax.axis_index("s")
        pltpu.sync_copy(z_hbm.at[pl.ds(sid * rpt, rpt)],
                        acc.at[pl.ds(sid * rpt, rpt)])

        @pl.when(sid == 0)
        def _():
            pltpu.sync_copy(z_hbm.at[pl.ds(rpt * NS, rem)],
                            acc.at[pl.ds(rpt * NS, rem)])

        plsc.subcore_barrier()
        base = sid * epc
        slots = ((idx_a, rows_a, sem_a), (idx_b, rows_b, sem_b))

        def start(slot, ch):
            idx_bufs, rows_v, sem = slot
            eb = base + ch * MCH_S
            pltpu.async_copy(vals_hbm.at[cid].at[pl.ds(eb, MCH_S)],
                             rows_v, sem)
            for kk in range(KSUB_S):
                pltpu.async_copy(dst_hbm.at[pl.ds(eb + kk * CH, CH)],
                                 idx_bufs[kk], sem)

        def fin_scatter(slot, ch):
            idx_bufs, rows_v, sem = slot
            eb = base + ch * MCH_S
            pltpu.make_async_copy(vals_hbm.at[cid].at[pl.ds(eb, MCH_S)],
                                  rows_v, sem).wait()
            for kk in range(KSUB_S):
                pltpu.make_async_copy(dst_hbm.at[pl.ds(eb + kk * CH, CH)],
                                      idx_bufs[kk], sem).wait()
            for kk in range(KSUB_S):
                pltpu.sync_copy(
                    rows_v.at[pl.ds(kk * CH, CH)],
                    acc.at[idx_bufs[kk]],
                    add=True,
                )

        start(slots[0], 0)

        @pl.loop(0, nmc - 1, step=2)
        def _(g):
            start(slots[1], g + 1)
            fin_scatter(slots[0], g)
            start(slots[0], g + 2)
            fin_scatter(slots[1], g + 1)

        fin_scatter(slots[0], nmc - 1)
        plsc.subcore_barrier()
        pltpu.sync_copy(acc.at[pl.ds(sid * rpt, rpt)],
                        out_hbm.at[cid].at[pl.ds(sid * rpt, rpt)])

        @pl.when(sid == 0)
        def _():
            pltpu.sync_copy(acc.at[pl.ds(rpt * NS, rem)],
                            out_hbm.at[cid].at[pl.ds(rpt * NS, rem)])

    return k(vals, dst, zeros)


def _tail(x, h, num, den, w1t, s1, q1, w2t, b2, lng, lnb, emit_h):
    def body(x_ref, h_ref, num_ref, den_ref, w1t_ref, s1_ref, q1_ref,
             w2t_ref, b2_ref, g_ref, bb_ref, xn_ref, *maybe_hn):
        dn = den_ref[...]
        safe = jnp.where(dn > 0.0, dn, 1.0)
        agg = jnp.where(dn > 0.0, num_ref[...] / safe, 0.0)
        out = agg + h_ref[...]
        z = jnp.dot(out, w1t_ref[...], preferred_element_type=jnp.float32)
        z = jnp.maximum(z * s1_ref[...] + q1_ref[...], 0.0)
        y = jnp.dot(z, w2t_ref[...], preferred_element_type=jnp.float32)
        xn = x_ref[...] + y + b2_ref[...]
        xn_ref[...] = xn
        if emit_h:
            mu = jnp.mean(xn, axis=1, keepdims=True)
            var = jnp.mean((xn - mu) ** 2, axis=1, keepdims=True)
            hn = (xn - mu) * lax.rsqrt(var + LN_EPS) * g_ref[...] + bb_ref[...]
            maybe_hn[0][...] = jnp.maximum(hn, 0.0)

    n_out = 2 if emit_h else 1
    row = lambda i: (i, 0)
    full = lambda i: (0, 0)
    res = pl.pallas_call(
        body,
        grid=(N // BN_ROWS,),
        in_specs=[
            pl.BlockSpec((BN_ROWS, D), row),
            pl.BlockSpec((BN_ROWS, D), row),
            pl.BlockSpec((BN_ROWS, D), row),
            pl.BlockSpec((BN_ROWS, D), row),
            pl.BlockSpec((D, 2 * D), full),
            pl.BlockSpec((1, 2 * D), full),
            pl.BlockSpec((1, 2 * D), full),
            pl.BlockSpec((2 * D, D), full),
            pl.BlockSpec((1, D), full),
            pl.BlockSpec((1, D), full),
            pl.BlockSpec((1, D), full),
        ],
        out_specs=[pl.BlockSpec((BN_ROWS, D), row)] * n_out,
        out_shape=[jax.ShapeDtypeStruct((N, D), jnp.float32)] * n_out,
    )(x, h, num, den, w1t, s1.reshape(1, 2 * D), q1.reshape(1, 2 * D),
      w2t, b2.reshape(1, D), lng.reshape(1, D), lnb.reshape(1, D))
    return (res[0], res[1]) if emit_h else (res[0], None)


def _pool_body(bid_ref, x_ref, fot_ref, fob_ref, o_ref, acc, cnt):
    i = pl.program_id(0)

    @pl.when(i == 0)
    def _():
        acc[...] = jnp.zeros_like(acc)
        cnt[...] = jnp.zeros_like(cnt)

    bids = bid_ref[0]
    gid = lax.broadcasted_iota(jnp.int32, (G, BN_ROWS), 0)
    oh = (gid == bids).astype(jnp.float32)
    acc[...] += jnp.dot(oh, x_ref[...], preferred_element_type=jnp.float32)
    cnt[...] = cnt[...] + jnp.sum(oh, axis=1, keepdims=True)

    @pl.when(i == pl.num_programs(0) - 1)
    def _():
        pooled = acc[...] / jnp.maximum(cnt[...], 1.0)
        pooled = jnp.maximum(pooled, 0.0)
        o_ref[...] = (
            jnp.dot(pooled, fot_ref[...], preferred_element_type=jnp.float32)
            + fob_ref[...]
        )


def _pool(batch3, x, fot, fob):
    return pl.pallas_call(
        _pool_body,
        grid=(N // BN_ROWS,),
        in_specs=[
            pl.BlockSpec((1, 1, BN_ROWS), lambda i: (i, 0, 0)),
            pl.BlockSpec((BN_ROWS, D), lambda i: (i, 0)),
            pl.BlockSpec((D, D), lambda i: (0, 0)),
            pl.BlockSpec((1, D), lambda i: (0, 0)),
        ],
        out_specs=pl.BlockSpec((G, D), lambda i: (0, 0)),
        out_shape=jax.ShapeDtypeStruct((G, D), jnp.float32),
        scratch_shapes=[
            pltpu.VMEM((G, D), jnp.float32),
            pltpu.VMEM((G, D), jnp.float32),
        ],
    )(batch3, x, fot, fob.reshape(1, D))


def kernel(x, edge_index, edge_attr, batch, fc_in_w, fc_in_b,
           ln_g0, ln_b0, lin_edge_w0, lin_edge_b0, t0, mlp_w10, mlp_b10,
           bn_g0, bn_b0, bn_rm0, bn_rv0, mlp_w20, mlp_b20,
           ln_g1, ln_b1, lin_edge_w1, lin_edge_b1, t1, mlp_w11, mlp_b11,
           bn_g1, bn_b1, bn_rm1, bn_rv1, mlp_w21, mlp_b21,
           fc_out_w, fc_out_b):
    src = edge_index[0]
    dst = edge_index[1]
    batch3 = batch.reshape(N // BN_ROWS, 1, BN_ROWS)
    zeros = jnp.zeros((N, D), jnp.float32)

    amax = _edge_attr_absmax(edge_attr)[0, 0]

    x1, h = _fcin_ln(x, fc_in_w.T, fc_in_b, ln_g0, ln_b0)

    layers = [
        (ln_g0, ln_b0, lin_edge_w0, lin_edge_b0, t0, mlp_w10, mlp_b10,
         bn_g0, bn_b0, bn_rm0, bn_rv0, mlp_w20, mlp_b20, ln_g1, ln_b1),
        (ln_g1, ln_b1, lin_edge_w1, lin_edge_b1, t1, mlp_w11, mlp_b11,
         bn_g1, bn_b1, bn_rm1, bn_rv1, mlp_w21, mlp_b21, None, None),
    ]
    for l, (lng, lnb, lew, leb, t, w1, b1, bg, bb, brm, brv, w2, b2,
            lng_next, lnb_next) in enumerate(layers):
        srow = jnp.max(jnp.sum(jnp.abs(lew), axis=1))
        bound = (11.32 * jnp.max(jnp.abs(lng)) + jnp.max(jnp.abs(lnb))
                 + amax * srow + jnp.max(jnp.abs(leb)))
        c = jnp.minimum(jnp.abs(t) * bound + 1.0, 80.0)

        gathered = _sc_gather(h, src)
        vals = _mid(t, c, gathered, edge_attr, lew.T, leb)
        nd = _sc_scatter(vals, dst, zeros)

        s1 = bg * lax.rsqrt(brv + BN_EPS)
        q1 = (b1 - brm) * s1 + bb
        if l == 0:
            x1, h = _tail(x1, h, nd[0], nd[1], w1.T, s1, q1, w2.T, b2,
                          lng_next, lnb_next, emit_h=True)
        else:
            x1, _ = _tail(x1, h, nd[0], nd[1], w1.T, s1, q1, w2.T, b2,
                          lng, lnb, emit_h=False)

    return _pool(batch3, x1, fc_out_w.T, fc_out_b)
